# baseline (speedup 1.0000x reference)
import functools

import jax
import jax.numpy as jnp
from jax import lax
from jax.experimental import pallas as pl
from jax.experimental.pallas import tpu as pltpu
from jax.experimental.pallas import tpu_sc as plsc

PASSES = 8
T = 4
D = 150
DP = 160
G = 16
NP = 10240
CH = 64
NTILES = 32
PER_TILE_ROWS = NP // 16


def _mm_relu_body(h_ref, w_ref, b_ref, o_ref):
    acc = jnp.dot(h_ref[...], w_ref[...], preferred_element_type=jnp.float32)
    o_ref[...] = jnp.maximum(acc + b_ref[...], 0.0)


def _gru_step(inc_ref, h_ref, wx_ref, uh_ref, bi_ref, bh_ref):
    x = inc_ref[0] + inc_ref[1]
    h = h_ref[...]
    gi = jnp.dot(x, wx_ref[...], preferred_element_type=jnp.float32) + bi_ref[...]
    gh = jnp.dot(h, uh_ref[...], preferred_element_type=jnp.float32) + bh_ref[...]
    r = jax.nn.sigmoid(gi[:, :DP] + gh[:, :DP])
    z = jax.nn.sigmoid(gi[:, DP:2 * DP] + gh[:, DP:2 * DP])
    n = jnp.tanh(gi[:, 2 * DP:] + r * gh[:, 2 * DP:])
    return (1.0 - z) * n + z * h


def _gru_msg_body(inc_ref, h_ref, wx_ref, uh_ref, bi_ref, bh_ref,
                  w_ref, b_ref, h_out_ref, msg_ref):
    hn = _gru_step(inc_ref, h_ref, wx_ref, uh_ref, bi_ref, bh_ref)
    h_out_ref[...] = hn
    acc = jnp.dot(hn, w_ref[...], preferred_element_type=jnp.float32)
    msg_ref[...] = jnp.maximum(acc + b_ref[...], 0.0)


def _leaky(x):
    return jnp.where(x > 0, x, 0.01 * x)


def _gru_readout_body(inc_ref, h_ref, wx_ref, uh_ref, bi_ref, bh_ref,
                      gid_ref, pc_ref, w1_ref, b1_ref, w2_ref, b2_ref,
                      wl_ref, bl_ref, o_ref, acc_ref, *, nsteps):
    i = pl.program_id(0)

    @pl.when(i == 0)
    def _():
        acc_ref[...] = jnp.zeros_like(acc_ref)

    hn = _gru_step(inc_ref, h_ref, wx_ref, uh_ref, bi_ref, bh_ref)
    gid = gid_ref[...]
    onehot = (gid == lax.broadcasted_iota(jnp.int32, (1, G), 1)).astype(jnp.float32)
    acc_ref[...] += lax.dot_general(onehot, hn, (((0,), (0,)), ((), ())),
                                    preferred_element_type=jnp.float32)

    @pl.when(i == nsteps - 1)
    def _():
        g = acc_ref[...]
        gl = jnp.log(g)
        gl = jnp.where(jnp.isnan(gl), 0.0, gl)
        gl = jnp.maximum(gl, 0.0)
        col = lax.broadcasted_iota(jnp.int32, (G, DP), 1)
        xin = jnp.where(col == D, pc_ref[...], gl)
        x1 = _leaky(jnp.dot(xin, w1_ref[...], preferred_element_type=jnp.float32) + b1_ref[...])
        x2 = _leaky(jnp.dot(x1, w2_ref[...], preferred_element_type=jnp.float32) + b2_ref[...])
        o_ref[...] = jnp.dot(x2, wl_ref[...], preferred_element_type=jnp.float32) + bl_ref[...]


def _make_sc_gather_scatter(nchunk):
    mesh = plsc.VectorSubcoreMesh(core_axis_name="c", subcore_axis_name="s")
    npair = nchunk // 2

    @functools.partial(
        pl.kernel,
        mesh=mesh,
        compiler_params=pltpu.CompilerParams(use_tc_tiling_on_sc=False),
        out_type=jax.ShapeDtypeStruct((2, NP, DP), jnp.float32),
        scratch_types=[
            pltpu.VMEM((nchunk, CH), jnp.int32),
            pltpu.VMEM((CH,), jnp.int32),
            pltpu.VMEM((CH,), jnp.int32),
            pltpu.VMEM((CH,), jnp.int32),
            pltpu.VMEM((CH,), jnp.int32),
            pltpu.VMEM((CH, DP), jnp.float32),
            pltpu.VMEM((CH, DP), jnp.float32),
            pltpu.VMEM_SHARED((NP, DP), jnp.float32),
            pltpu.SemaphoreType.DMA,
            pltpu.SemaphoreType.DMA,
        ],
    )
    def sc_kernel(pidx_hbm, msg_hbm, zeros_hbm, out_hbm,
                  pidx_v, ib0, ib1, db0, db1, rows0, rows1, acc_sh, g0, g1):
        c = lax.axis_index("c")
        s = lax.axis_index("s")

        def unpack(j, ib, db):
            for k in range(CH // 16):
                v = pidx_v[j, pl.ds(16 * k, 16)]
                db[pl.ds(16 * k, 16)] = lax.shift_right_logical(v, 16)
                ib[pl.ds(16 * k, 16)] = lax.bitwise_and(v, 0xFFFF)

        def wait_g(buf, sem):
            pltpu.make_async_copy(msg_hbm.at[ib0], buf, sem).wait()

        pltpu.sync_copy(pidx_hbm.at[c, s], pidx_v)
        unpack(0, ib0, db0)
        pltpu.async_copy(msg_hbm.at[ib0], rows0, g0)
        unpack(1, ib1, db1)
        pltpu.async_copy(msg_hbm.at[ib1], rows1, g1)
        pltpu.sync_copy(zeros_hbm, acc_sh.at[pl.ds(s * PER_TILE_ROWS, PER_TILE_ROWS)])
        plsc.subcore_barrier()

        def pair(p, carry):
            j0 = 2 * p
            wait_g(rows0, g0)
            pltpu.sync_copy(rows0, acc_sh.at[db0], add=True)
            unpack(j0 + 2, ib0, db0)
            pltpu.async_copy(msg_hbm.at[ib0], rows0, g0)
            wait_g(rows1, g1)
            pltpu.sync_copy(rows1, acc_sh.at[db1], add=True)
            unpack(j0 + 3, ib1, db1)
            pltpu.async_copy(msg_hbm.at[ib1], rows1, g1)
            return carry

        lax.fori_loop(0, npair - 1, pair, 0)
        wait_g(rows0, g0)
        pltpu.sync_copy(rows0, acc_sh.at[db0], add=True)
        wait_g(rows1, g1)
        pltpu.sync_copy(rows1, acc_sh.at[db1], add=True)
        plsc.subcore_barrier()
        pltpu.sync_copy(acc_sh.at[pl.ds(s * PER_TILE_ROWS, PER_TILE_ROWS)],
                        out_hbm.at[c, pl.ds(s * PER_TILE_ROWS, PER_TILE_ROWS)])

    return sc_kernel


def kernel(nodes, problemClass, edge_index, edge_type, graph_ids, edgeW, edgeB,
           gru_Wih, gru_Whh, gru_bih, gru_bhh, fc1W, fc1b, fc2W, fc2b, fcLW, fcLb):
    N = nodes.shape[0]
    E = edge_index.shape[1]
    BLK = 512
    nsteps = NP // BLK

    h0 = jnp.pad(nodes, ((0, NP - N), (0, DP - D)))
    w = jnp.pad(edgeW, ((0, 0), (0, DP - D), (0, DP - D)))
    W_all = jnp.transpose(w, (2, 0, 1)).reshape(DP, T * DP)
    b_all = jnp.pad(edgeB, ((0, 0), (0, DP - D))).reshape(1, T * DP)

    def _gate_pack(m):
        m3 = m.reshape(3, D, D)
        m3 = jnp.transpose(m3, (2, 0, 1))
        m3 = jnp.pad(m3, ((0, DP - D), (0, 0), (0, DP - D)))
        return m3.reshape(DP, 3 * DP)

    Wx = _gate_pack(gru_Wih)
    Uh = _gate_pack(gru_Whh)
    bi = jnp.pad(gru_bih.reshape(3, D), ((0, 0), (0, DP - D))).reshape(1, 3 * DP)
    bh = jnp.pad(gru_bhh.reshape(3, D), ((0, 0), (0, DP - D))).reshape(1, 3 * DP)

    nchunk = -(-E // (NTILES * CH))
    nchunk += nchunk % 2
    nchunk = max(nchunk, 4)
    EP = NTILES * CH * nchunk
    src = edge_index[0]
    dst = edge_index[1]
    comb = src * T + edge_type
    comb = jnp.pad(comb, (0, EP - E), constant_values=N * T)
    dstp = jnp.pad(dst, (0, EP - E), constant_values=N)
    pidx3 = jnp.bitwise_or(comb, dstp << 16).reshape(2, 16, nchunk, CH)
    zeros_hbm = jnp.zeros((PER_TILE_ROWS, DP), jnp.float32)

    sc_gather_scatter = _make_sc_gather_scatter(nchunk)

    mm_relu = pl.pallas_call(
        _mm_relu_body,
        grid=(nsteps,),
        in_specs=[pl.BlockSpec((BLK, DP), lambda i: (i, 0)),
                  pl.BlockSpec((DP, T * DP), lambda i: (0, 0)),
                  pl.BlockSpec((1, T * DP), lambda i: (0, 0))],
        out_specs=pl.BlockSpec((BLK, T * DP), lambda i: (i, 0)),
        out_shape=jax.ShapeDtypeStruct((NP, T * DP), jnp.float32),
    )

    gru_specs = [pl.BlockSpec((2, BLK, DP), lambda i: (0, i, 0)),
                 pl.BlockSpec((BLK, DP), lambda i: (i, 0)),
                 pl.BlockSpec((DP, 3 * DP), lambda i: (0, 0)),
                 pl.BlockSpec((DP, 3 * DP), lambda i: (0, 0)),
                 pl.BlockSpec((1, 3 * DP), lambda i: (0, 0)),
                 pl.BlockSpec((1, 3 * DP), lambda i: (0, 0))]

    gru_msg = pl.pallas_call(
        _gru_msg_body,
        grid=(nsteps,),
        in_specs=gru_specs + [pl.BlockSpec((DP, T * DP), lambda i: (0, 0)),
                              pl.BlockSpec((1, T * DP), lambda i: (0, 0))],
        out_specs=[pl.BlockSpec((BLK, DP), lambda i: (i, 0)),
                   pl.BlockSpec((BLK, T * DP), lambda i: (i, 0))],
        out_shape=[jax.ShapeDtypeStruct((NP, DP), jnp.float32),
                   jax.ShapeDtypeStruct((NP, T * DP), jnp.float32)],
    )

    gid = jnp.pad(graph_ids, (0, NP - N), constant_values=G).reshape(NP, 1)
    w1 = jnp.pad(fc1W.T, ((0, DP - (D + 1)), (0, 0)))
    b1 = fc1b.reshape(1, 80)
    w2 = fc2W.T
    b2 = fc2b.reshape(1, 80)
    wl = fcLW.T
    bl = fcLb.reshape(1, 10)

    gru_readout = pl.pallas_call(
        functools.partial(_gru_readout_body, nsteps=nsteps),
        grid=(nsteps,),
        in_specs=gru_specs + [pl.BlockSpec((BLK, 1), lambda i: (i, 0)),
                              pl.BlockSpec((G, 1), lambda i: (0, 0)),
                              pl.BlockSpec((DP, 80), lambda i: (0, 0)),
                              pl.BlockSpec((1, 80), lambda i: (0, 0)),
                              pl.BlockSpec((80, 80), lambda i: (0, 0)),
                              pl.BlockSpec((1, 80), lambda i: (0, 0)),
                              pl.BlockSpec((80, 10), lambda i: (0, 0)),
                              pl.BlockSpec((1, 10), lambda i: (0, 0))],
        out_specs=pl.BlockSpec((G, 10), lambda i: (0, 0)),
        out_shape=jax.ShapeDtypeStruct((G, 10), jnp.float32),
        scratch_shapes=[pltpu.VMEM((G, DP), jnp.float32)],
    )

    h = h0
    per_type = mm_relu(h, W_all, b_all)
    for _ in range(PASSES - 1):
        inc = sc_gather_scatter(pidx3, per_type.reshape(NP * T, DP), zeros_hbm)
        h, per_type = gru_msg(inc, h, Wx, Uh, bi, bh, W_all, b_all)
    inc = sc_gather_scatter(pidx3, per_type.reshape(NP * T, DP), zeros_hbm)
    out = gru_readout(inc, h, Wx, Uh, bi, bh,
                      gid, problemClass, w1, b1, w2, b2, wl, bl)
    return out


# speedup vs baseline: 1.4313x; 1.4313x over previous
import functools

import jax
import jax.numpy as jnp
from jax import lax
from jax.experimental import pallas as pl
from jax.experimental.pallas import tpu as pltpu
from jax.experimental.pallas import tpu_sc as plsc

PASSES = 8
T = 4
D = 150
DP = 160
G = 16
NP = 10240
CH = 56
NTILES = 32
PER_TILE_ROWS = NP // 16


def _mm_relu_body(h_ref, w_ref, b_ref, o_ref):
    acc = jnp.dot(h_ref[...], w_ref[...], preferred_element_type=jnp.float32)
    o_ref[...] = jnp.maximum(acc + b_ref[...], 0.0)


def _gru_step(inc_ref, h_ref, wx_ref, uh_ref, bi_ref, bh_ref):
    x = inc_ref[0] + inc_ref[1]
    h = h_ref[...]
    gi = jnp.dot(x, wx_ref[...], preferred_element_type=jnp.float32) + bi_ref[...]
    gh = jnp.dot(h, uh_ref[...], preferred_element_type=jnp.float32) + bh_ref[...]
    r = jax.nn.sigmoid(gi[:, :DP] + gh[:, :DP])
    z = jax.nn.sigmoid(gi[:, DP:2 * DP] + gh[:, DP:2 * DP])
    n = jnp.tanh(gi[:, 2 * DP:] + r * gh[:, 2 * DP:])
    return (1.0 - z) * n + z * h


def _gru_msg_body(inc_ref, h_ref, wx_ref, uh_ref, bi_ref, bh_ref,
                  w_ref, b_ref, h_out_ref, msg_ref):
    hn = _gru_step(inc_ref, h_ref, wx_ref, uh_ref, bi_ref, bh_ref)
    h_out_ref[...] = hn
    acc = jnp.dot(hn, w_ref[...], preferred_element_type=jnp.float32)
    msg_ref[...] = jnp.maximum(acc + b_ref[...], 0.0)


def _leaky(x):
    return jnp.where(x > 0, x, 0.01 * x)


def _gru_readout_body(inc_ref, h_ref, wx_ref, uh_ref, bi_ref, bh_ref,
                      gid_ref, pc_ref, w1_ref, b1_ref, w2_ref, b2_ref,
                      wl_ref, bl_ref, o_ref, acc_ref, *, nsteps):
    i = pl.program_id(0)

    @pl.when(i == 0)
    def _():
        acc_ref[...] = jnp.zeros_like(acc_ref)

    hn = _gru_step(inc_ref, h_ref, wx_ref, uh_ref, bi_ref, bh_ref)
    gid = gid_ref[...]
    onehot = (gid == lax.broadcasted_iota(jnp.int32, (1, G), 1)).astype(jnp.float32)
    acc_ref[...] += lax.dot_general(onehot, hn, (((0,), (0,)), ((), ())),
                                    preferred_element_type=jnp.float32)

    @pl.when(i == nsteps - 1)
    def _():
        g = acc_ref[...]
        gl = jnp.log(g)
        gl = jnp.where(jnp.isnan(gl), 0.0, gl)
        gl = jnp.maximum(gl, 0.0)
        col = lax.broadcasted_iota(jnp.int32, (G, DP), 1)
        xin = jnp.where(col == D, pc_ref[...], gl)
        x1 = _leaky(jnp.dot(xin, w1_ref[...], preferred_element_type=jnp.float32) + b1_ref[...])
        x2 = _leaky(jnp.dot(x1, w2_ref[...], preferred_element_type=jnp.float32) + b2_ref[...])
        o_ref[...] = jnp.dot(x2, wl_ref[...], preferred_element_type=jnp.float32) + bl_ref[...]


def _make_sc_gather_scatter(nchunk):
    mesh = plsc.VectorSubcoreMesh(core_axis_name="c", subcore_axis_name="s")
    npair = nchunk // 2

    @functools.partial(
        pl.kernel,
        mesh=mesh,
        compiler_params=pltpu.CompilerParams(use_tc_tiling_on_sc=False),
        out_type=jax.ShapeDtypeStruct((2, NP, DP), jnp.float32),
        scratch_types=[
            pltpu.VMEM((nchunk, CH), jnp.int32),
            pltpu.VMEM((nchunk, CH), jnp.int32),
            pltpu.VMEM((CH, DP), jnp.float32),
            pltpu.VMEM((CH, DP), jnp.float32),
            pltpu.VMEM_SHARED((NP, DP), jnp.float32),
            pltpu.SemaphoreType.DMA,
            pltpu.SemaphoreType.DMA,
        ],
    )
    def sc_kernel(cidx_hbm, dst_hbm, msg_hbm, zeros_hbm, out_hbm,
                  idx_v, dst_v, rows0, rows1, acc_sh, sem0, sem1):
        c = lax.axis_index("c")
        s = lax.axis_index("s")
        pltpu.sync_copy(zeros_hbm, acc_sh.at[pl.ds(s * PER_TILE_ROWS, PER_TILE_ROWS)])
        plsc.subcore_barrier()
        pltpu.sync_copy(cidx_hbm.at[c, s], idx_v)
        pltpu.sync_copy(dst_hbm.at[c, s], dst_v)
        pltpu.async_copy(msg_hbm.at[idx_v.at[0]], rows0, sem0)
        pltpu.async_copy(msg_hbm.at[idx_v.at[1]], rows1, sem1)

        def pair(p, carry):
            j0 = 2 * p
            pltpu.make_async_copy(msg_hbm.at[idx_v.at[0]], rows0, sem0).wait()
            pltpu.sync_copy(rows0, acc_sh.at[dst_v.at[j0]], add=True)
            pltpu.async_copy(msg_hbm.at[idx_v.at[j0 + 2]], rows0, sem0)
            pltpu.make_async_copy(msg_hbm.at[idx_v.at[0]], rows1, sem1).wait()
            pltpu.sync_copy(rows1, acc_sh.at[dst_v.at[j0 + 1]], add=True)
            pltpu.async_copy(msg_hbm.at[idx_v.at[j0 + 3]], rows1, sem1)
            return carry

        lax.fori_loop(0, npair - 1, pair, 0)
        pltpu.make_async_copy(msg_hbm.at[idx_v.at[0]], rows0, sem0).wait()
        pltpu.sync_copy(rows0, acc_sh.at[dst_v.at[nchunk - 2]], add=True)
        pltpu.make_async_copy(msg_hbm.at[idx_v.at[0]], rows1, sem1).wait()
        pltpu.sync_copy(rows1, acc_sh.at[dst_v.at[nchunk - 1]], add=True)
        plsc.subcore_barrier()
        pltpu.sync_copy(acc_sh.at[pl.ds(s * PER_TILE_ROWS, PER_TILE_ROWS)],
                        out_hbm.at[c, pl.ds(s * PER_TILE_ROWS, PER_TILE_ROWS)])

    return sc_kernel


def kernel(nodes, problemClass, edge_index, edge_type, graph_ids, edgeW, edgeB,
           gru_Wih, gru_Whh, gru_bih, gru_bhh, fc1W, fc1b, fc2W, fc2b, fcLW, fcLb):
    N = nodes.shape[0]
    E = edge_index.shape[1]
    BLK = 512
    nsteps = NP // BLK

    h0 = jnp.pad(nodes, ((0, NP - N), (0, DP - D)))
    w = jnp.pad(edgeW, ((0, 0), (0, DP - D), (0, DP - D)))
    W_all = jnp.transpose(w, (2, 0, 1)).reshape(DP, T * DP)
    b_all = jnp.pad(edgeB, ((0, 0), (0, DP - D))).reshape(1, T * DP)

    def _gate_pack(m):
        m3 = m.reshape(3, D, D)
        m3 = jnp.transpose(m3, (2, 0, 1))
        m3 = jnp.pad(m3, ((0, DP - D), (0, 0), (0, DP - D)))
        return m3.reshape(DP, 3 * DP)

    Wx = _gate_pack(gru_Wih)
    Uh = _gate_pack(gru_Whh)
    bi = jnp.pad(gru_bih.reshape(3, D), ((0, 0), (0, DP - D))).reshape(1, 3 * DP)
    bh = jnp.pad(gru_bhh.reshape(3, D), ((0, 0), (0, DP - D))).reshape(1, 3 * DP)

    nchunk = -(-E // (NTILES * CH))
    nchunk += nchunk % 2
    nchunk = max(nchunk, 4)
    EP = NTILES * CH * nchunk
    src = edge_index[0]
    dst = edge_index[1]
    comb = src * T + edge_type
    comb = jnp.pad(comb, (0, EP - E), constant_values=N * T)
    dstp = jnp.pad(dst, (0, EP - E), constant_values=N)
    cidx3 = comb.reshape(2, 16, nchunk, CH)
    dst3 = dstp.reshape(2, 16, nchunk, CH)
    zeros_hbm = jnp.zeros((PER_TILE_ROWS, DP), jnp.float32)

    sc_gather_scatter = _make_sc_gather_scatter(nchunk)

    mm_relu = pl.pallas_call(
        _mm_relu_body,
        grid=(nsteps,),
        in_specs=[pl.BlockSpec((BLK, DP), lambda i: (i, 0)),
                  pl.BlockSpec((DP, T * DP), lambda i: (0, 0)),
                  pl.BlockSpec((1, T * DP), lambda i: (0, 0))],
        out_specs=pl.BlockSpec((BLK, T * DP), lambda i: (i, 0)),
        out_shape=jax.ShapeDtypeStruct((NP, T * DP), jnp.float32),
    )

    gru_specs = [pl.BlockSpec((2, BLK, DP), lambda i: (0, i, 0)),
                 pl.BlockSpec((BLK, DP), lambda i: (i, 0)),
                 pl.BlockSpec((DP, 3 * DP), lambda i: (0, 0)),
                 pl.BlockSpec((DP, 3 * DP), lambda i: (0, 0)),
                 pl.BlockSpec((1, 3 * DP), lambda i: (0, 0)),
                 pl.BlockSpec((1, 3 * DP), lambda i: (0, 0))]

    gru_msg = pl.pallas_call(
        _gru_msg_body,
        grid=(nsteps,),
        in_specs=gru_specs + [pl.BlockSpec((DP, T * DP), lambda i: (0, 0)),
                              pl.BlockSpec((1, T * DP), lambda i: (0, 0))],
        out_specs=[pl.BlockSpec((BLK, DP), lambda i: (i, 0)),
                   pl.BlockSpec((BLK, T * DP), lambda i: (i, 0))],
        out_shape=[jax.ShapeDtypeStruct((NP, DP), jnp.float32),
                   jax.ShapeDtypeStruct((NP, T * DP), jnp.float32)],
    )

    gid = jnp.pad(graph_ids, (0, NP - N), constant_values=G).reshape(NP, 1)
    w1 = jnp.pad(fc1W.T, ((0, DP - (D + 1)), (0, 0)))
    b1 = fc1b.reshape(1, 80)
    w2 = fc2W.T
    b2 = fc2b.reshape(1, 80)
    wl = fcLW.T
    bl = fcLb.reshape(1, 10)

    gru_readout = pl.pallas_call(
        functools.partial(_gru_readout_body, nsteps=nsteps),
        grid=(nsteps,),
        in_specs=gru_specs + [pl.BlockSpec((BLK, 1), lambda i: (i, 0)),
                              pl.BlockSpec((G, 1), lambda i: (0, 0)),
                              pl.BlockSpec((DP, 80), lambda i: (0, 0)),
                              pl.BlockSpec((1, 80), lambda i: (0, 0)),
                              pl.BlockSpec((80, 80), lambda i: (0, 0)),
                              pl.BlockSpec((1, 80), lambda i: (0, 0)),
                              pl.BlockSpec((80, 10), lambda i: (0, 0)),
                              pl.BlockSpec((1, 10), lambda i: (0, 0))],
        out_specs=pl.BlockSpec((G, 10), lambda i: (0, 0)),
        out_shape=jax.ShapeDtypeStruct((G, 10), jnp.float32),
        scratch_shapes=[pltpu.VMEM((G, DP), jnp.float32)],
    )

    h = h0
    per_type = mm_relu(h, W_all, b_all)
    for _ in range(PASSES - 1):
        inc = sc_gather_scatter(cidx3, dst3, per_type.reshape(NP * T, DP), zeros_hbm)
        h, per_type = gru_msg(inc, h, Wx, Uh, bi, bh, W_all, b_all)
    inc = sc_gather_scatter(cidx3, dst3, per_type.reshape(NP * T, DP), zeros_hbm)
    out = gru_readout(inc, h, Wx, Uh, bi, bh,
                      gid, problemClass, w1, b1, w2, b2, wl, bl)
    return out
